# Initial kernel scaffold; baseline (speedup 1.0000x reference)
import jax
import jax.numpy as jnp
from jax.experimental import pallas as pl


def kernel(W):
    raise NotImplementedError("write your pallas kernel here")


# speedup vs baseline: 9.3240x; 9.3240x over previous
import jax
import jax.numpy as jnp
import numpy as np
from jax.experimental import pallas as pl

V_BIT = 8
V_PI = 4.36
V_MAX = 10.8
GAMMA = np.pi / (V_PI ** 2)
TWO_PI = 2.0 * np.pi
NLEV = float(2 ** V_BIT - 1)
V_2PI = np.sqrt(TWO_PI / GAMMA)
N = 256


def _quant_kernel(w_ref, o_ref):
    w = w_ref[...]
    v = jnp.sqrt(jnp.mod(w, TWO_PI) / GAMMA)
    x = v / V_MAX
    xq = jnp.round(x * NLEV) / NLEV
    v_q = xq * V_MAX
    v_q = jnp.where(v_q < V_2PI, v_q, 0.0)
    ph = jnp.mod(GAMMA * v_q * v_q, TWO_PI)
    ph = jnp.where(ph > np.pi, ph - TWO_PI, ph)
    r = jax.lax.broadcasted_iota(jnp.int32, w.shape, w.ndim - 2)
    c = jax.lax.broadcasted_iota(jnp.int32, w.shape, w.ndim - 1)
    o_ref[...] = jnp.where(r + c <= N - 2, ph, 0.0)


def kernel(W):
    b = W.shape[0]
    blk = 8
    return pl.pallas_call(
        _quant_kernel,
        grid=(b // blk,),
        in_specs=[pl.BlockSpec((blk, N, N), lambda i: (i, 0, 0))],
        out_specs=pl.BlockSpec((blk, N, N), lambda i: (i, 0, 0)),
        out_shape=jax.ShapeDtypeStruct(W.shape, W.dtype),
    )(W)
